# Initial kernel scaffold; baseline (speedup 1.0000x reference)
import functools

import jax
import jax.numpy as jnp
from jax import lax
from jax.experimental import pallas as pl
from jax.experimental.pallas import tpu as pltpu
from jax.experimental.pallas import tpu_sc as plsc

THRESH = 0.5
CODE_STRIDE = 32


def _argmax_pack_body(cls_num, out_ref, tgt_ref, code_ref):
    oidx = jnp.zeros(out_ref.shape[2:], jnp.int32)
    for c in range(cls_num - 1, -1, -1):
        oidx = jnp.where(out_ref[0, c] > THRESH, c, oidx)
    bestv = tgt_ref[0, 0]
    tidx = jnp.zeros(tgt_ref.shape[2:], jnp.int32)
    for c in range(1, cls_num):
        v = tgt_ref[0, c]
        m = v > bestv
        tidx = jnp.where(m, c, tidx)
        bestv = jnp.maximum(bestv, v)
    code_ref[0] = tidx * CODE_STRIDE + oidx


def _argmax_pack(output, target, hb=64):
    b, cls_num, h, w = output.shape
    grid = (b, h // hb)
    return pl.pallas_call(
        functools.partial(_argmax_pack_body, cls_num),
        grid=grid,
        in_specs=[
            pl.BlockSpec((1, cls_num, hb, w), lambda i, j: (i, 0, j, 0)),
            pl.BlockSpec((1, cls_num, hb, w), lambda i, j: (i, 0, j, 0)),
        ],
        out_specs=pl.BlockSpec((1, hb, w), lambda i, j: (i, j, 0)),
        out_shape=jax.ShapeDtypeStruct((b, h, w), jnp.int32),
        compiler_params=pltpu.CompilerParams(
            dimension_semantics=("parallel", "parallel"),
        ),
    )(output, target)


def _sc_hist(codes, nbins):
    info = plsc.get_sparse_core_info()
    nc, ns, lanes = info.num_cores, info.num_subcores, info.num_lanes
    nw = nc * ns
    (p,) = codes.shape
    per_w = p // nw
    chunk = 4096
    nchunks = per_w // chunk
    assert per_w % chunk == 0 and p % nw == 0

    mesh = plsc.VectorSubcoreMesh(core_axis_name="c", subcore_axis_name="s")

    @functools.partial(
        pl.kernel,
        mesh=mesh,
        out_type=jax.ShapeDtypeStruct((nw, nbins), jnp.int32),
        scratch_types=[
            pltpu.VMEM((chunk,), jnp.int32),
            pltpu.VMEM((chunk,), jnp.int32),
            pltpu.VMEM((lanes * nbins,), jnp.int32),
            pltpu.VMEM((nbins,), jnp.int32),
            pltpu.SemaphoreType.DMA,
            pltpu.SemaphoreType.DMA,
        ],
    )
    def hist_kernel(codes_hbm, out_hbm, buf0, buf1, lhist, merged, sem0, sem1):
        wid = lax.axis_index("s") * nc + lax.axis_index("c")
        base = wid * per_w

        zeros16 = jnp.zeros((lanes,), jnp.int32)

        def zero_body(j, _):
            lhist[pl.ds(j * lanes, lanes)] = zeros16
            return 0

        lax.fori_loop(0, (lanes * nbins) // lanes, zero_body, 0)

        ones = jnp.ones((lanes,), jnp.int32)
        lane_base = lax.iota(jnp.int32, lanes) * nbins

        bufs = (buf0, buf1)
        sems = (sem0, sem1)

        def consume(buf):
            def body(g, _):
                v = buf[pl.ds(g * lanes, lanes)]
                plsc.addupdate_scatter(lhist, [lane_base + v], ones)
                return 0

            lax.fori_loop(0, chunk // lanes, body, 0)

        copies = [
            pltpu.async_copy(codes_hbm.at[pl.ds(base, chunk)], bufs[0], sems[0]),
            None,
        ]
        for i in range(nchunks):
            if i + 1 < nchunks:
                copies[(i + 1) % 2] = pltpu.async_copy(
                    codes_hbm.at[pl.ds(base + (i + 1) * chunk, chunk)],
                    bufs[(i + 1) % 2],
                    sems[(i + 1) % 2],
                )
            copies[i % 2].wait()
            consume(bufs[i % 2])

        for j in range(nbins // lanes):
            acc = jnp.zeros((lanes,), jnp.int32)
            for lane in range(lanes):
                acc = acc + lhist[pl.ds(lane * nbins + j * lanes, lanes)]
            merged[pl.ds(j * lanes, lanes)] = acc

        pltpu.sync_copy(merged, out_hbm.at[wid])

    return hist_kernel(codes)


def kernel(output, target):
    b, cls_num, h, w = output.shape
    nbins = cls_num * CODE_STRIDE
    codes = _argmax_pack(output, target)
    parts = _sc_hist(codes.reshape(-1), nbins)

    c = parts.sum(0).reshape(cls_num, CODE_STRIDE)[:, :cls_num]
    h_tgt = c.sum(1)
    h_out = c.sum(0)
    m = jnp.diagonal(c)
    tps = m.astype(jnp.float32)
    denom = (h_out + h_tgt - m).astype(jnp.float32)
    score = jnp.where(denom == 0, 0.0, tps / jnp.where(denom == 0, 1.0, denom))
    iou = score.mean()
    n = b * h * w
    total = b * cls_num * h * w
    t_sum = m.sum().astype(jnp.float32)
    pix_acc = (2.0 * t_sum - 2.0 * float(n) + float(total)) / float(total)
    return (iou, pix_acc)


# speedup vs baseline: 3.7611x; 3.7611x over previous
import functools

import jax
import jax.numpy as jnp
from jax import lax
from jax.experimental import pallas as pl
from jax.experimental.pallas import tpu as pltpu
from jax.experimental.pallas import tpu_sc as plsc

THRESH = 0.5
CODE_STRIDE = 32


def _argmax_pack_body(cls_num, out_ref, tgt_ref, code_ref):
    oidx = jnp.zeros(out_ref.shape[2:], jnp.int32)
    for c in range(cls_num - 1, -1, -1):
        oidx = jnp.where(out_ref[0, c] > THRESH, c, oidx)
    bestv = tgt_ref[0, 0]
    tidx = jnp.zeros(tgt_ref.shape[2:], jnp.int32)
    for c in range(1, cls_num):
        v = tgt_ref[0, c]
        m = v > bestv
        tidx = jnp.where(m, c, tidx)
        bestv = jnp.maximum(bestv, v)
    code_ref[0] = tidx * CODE_STRIDE + oidx


def _argmax_pack(output, target, hb=64):
    b, cls_num, h, w = output.shape
    grid = (b, h // hb)
    return pl.pallas_call(
        functools.partial(_argmax_pack_body, cls_num),
        grid=grid,
        in_specs=[
            pl.BlockSpec((1, cls_num, hb, w), lambda i, j: (i, 0, j, 0)),
            pl.BlockSpec((1, cls_num, hb, w), lambda i, j: (i, 0, j, 0)),
        ],
        out_specs=pl.BlockSpec((1, hb, w), lambda i, j: (i, j, 0)),
        out_shape=jax.ShapeDtypeStruct((b, h, w), jnp.int32),
        compiler_params=pltpu.CompilerParams(
            dimension_semantics=("parallel", "parallel"),
        ),
    )(output, target)


def _sc_hist(codes, nbins):
    info = plsc.get_sparse_core_info()
    nc, ns, lanes = info.num_cores, info.num_subcores, info.num_lanes
    nw = nc * ns
    (p,) = codes.shape
    per_w = p // nw
    chunk = 4096
    nchunks = per_w // chunk
    assert per_w % chunk == 0 and p % nw == 0

    mesh = plsc.VectorSubcoreMesh(core_axis_name="c", subcore_axis_name="s")

    @functools.partial(
        pl.kernel,
        mesh=mesh,
        out_type=jax.ShapeDtypeStruct((nw, nbins), jnp.int32),
        scratch_types=[
            pltpu.VMEM((chunk,), jnp.int32),
            pltpu.VMEM((chunk,), jnp.int32),
            pltpu.VMEM((lanes * nbins,), jnp.int32),
            pltpu.VMEM((nbins,), jnp.int32),
            pltpu.SemaphoreType.DMA,
            pltpu.SemaphoreType.DMA,
        ],
        compiler_params=pltpu.CompilerParams(needs_layout_passes=False),
    )
    def hist_kernel(codes_hbm, out_hbm, buf0, buf1, lhist, merged, sem0, sem1):
        wid = lax.axis_index("s") * nc + lax.axis_index("c")
        base = wid * per_w

        zeros16 = jnp.zeros((lanes,), jnp.int32)

        def zero_body(j, _):
            lhist[pl.ds(j * lanes, lanes)] = zeros16
            return 0

        lax.fori_loop(0, (lanes * nbins) // lanes, zero_body, 0)

        ones = jnp.ones((lanes,), jnp.int32)
        lane_base = lax.iota(jnp.int32, lanes) * nbins

        bufs = (buf0, buf1)
        sems = (sem0, sem1)

        def consume(buf):
            def body(g, _):
                v = buf[pl.ds(g * lanes, lanes)]
                plsc.addupdate_scatter(lhist, [lane_base + v], ones)
                return 0

            lax.fori_loop(0, chunk // lanes, body, 0)

        copies = [
            pltpu.async_copy(codes_hbm.at[pl.ds(base, chunk)], bufs[0], sems[0]),
            None,
        ]
        for i in range(nchunks):
            if i + 1 < nchunks:
                copies[(i + 1) % 2] = pltpu.async_copy(
                    codes_hbm.at[pl.ds(base + (i + 1) * chunk, chunk)],
                    bufs[(i + 1) % 2],
                    sems[(i + 1) % 2],
                )
            copies[i % 2].wait()
            consume(bufs[i % 2])

        for j in range(nbins // lanes):
            acc = jnp.zeros((lanes,), jnp.int32)
            for lane in range(lanes):
                acc = acc + lhist[pl.ds(lane * nbins + j * lanes, lanes)]
            merged[pl.ds(j * lanes, lanes)] = acc

        pltpu.sync_copy(merged, out_hbm.at[wid])

    return hist_kernel(codes)


def kernel(output, target):
    b, cls_num, h, w = output.shape
    nbins = cls_num * CODE_STRIDE
    codes = _argmax_pack(output, target)
    parts = _sc_hist(codes.reshape(-1), nbins)

    c = parts.sum(0).reshape(cls_num, CODE_STRIDE)[:, :cls_num]
    h_tgt = c.sum(1)
    h_out = c.sum(0)
    m = jnp.diagonal(c)
    tps = m.astype(jnp.float32)
    denom = (h_out + h_tgt - m).astype(jnp.float32)
    score = jnp.where(denom == 0, 0.0, tps / jnp.where(denom == 0, 1.0, denom))
    iou = score.mean()
    n = b * h * w
    total = b * cls_num * h * w
    t_sum = m.sum().astype(jnp.float32)
    pix_acc = (2.0 * t_sum - 2.0 * float(n) + float(total)) / float(total)
    return (iou, pix_acc)


# speedup vs baseline: 3.8521x; 1.0242x over previous
import functools

import jax
import jax.numpy as jnp
from jax import lax
from jax.experimental import pallas as pl
from jax.experimental.pallas import tpu as pltpu
from jax.experimental.pallas import tpu_sc as plsc

THRESH = 0.5
CODE_STRIDE = 32


def _argmax_pack_body(cls_num, out_ref, tgt_ref, code_ref):
    oidx = jnp.zeros(out_ref.shape[2:], jnp.int32)
    for c in range(cls_num - 1, -1, -1):
        oidx = jnp.where(out_ref[0, c] > THRESH, c, oidx)
    bestv = tgt_ref[0, 0]
    tidx = jnp.zeros(tgt_ref.shape[2:], jnp.int32)
    for c in range(1, cls_num):
        v = tgt_ref[0, c]
        m = v > bestv
        tidx = jnp.where(m, c, tidx)
        bestv = jnp.maximum(bestv, v)
    code_ref[0] = tidx * CODE_STRIDE + oidx


def _argmax_pack(output, target, hb=128):
    b, cls_num, h, w = output.shape
    grid = (b, h // hb)
    return pl.pallas_call(
        functools.partial(_argmax_pack_body, cls_num),
        grid=grid,
        in_specs=[
            pl.BlockSpec((1, cls_num, hb, w), lambda i, j: (i, 0, j, 0)),
            pl.BlockSpec((1, cls_num, hb, w), lambda i, j: (i, 0, j, 0)),
        ],
        out_specs=pl.BlockSpec((1, hb, w), lambda i, j: (i, j, 0)),
        out_shape=jax.ShapeDtypeStruct((b, h, w), jnp.int32),
        compiler_params=pltpu.CompilerParams(
            dimension_semantics=("parallel", "parallel"),
        ),
    )(output, target)


def _sc_hist(codes, nbins):
    info = plsc.get_sparse_core_info()
    nc, ns, lanes = info.num_cores, info.num_subcores, info.num_lanes
    nw = nc * ns
    (p,) = codes.shape
    per_w = p // nw
    chunk = 4096
    nchunks = per_w // chunk
    assert per_w % chunk == 0 and p % nw == 0

    mesh = plsc.VectorSubcoreMesh(core_axis_name="c", subcore_axis_name="s")

    @functools.partial(
        pl.kernel,
        mesh=mesh,
        out_type=jax.ShapeDtypeStruct((nw, nbins), jnp.int32),
        scratch_types=[
            pltpu.VMEM((chunk,), jnp.int32),
            pltpu.VMEM((chunk,), jnp.int32),
            pltpu.VMEM((lanes * nbins,), jnp.int32),
            pltpu.VMEM((nbins,), jnp.int32),
            pltpu.SemaphoreType.DMA,
            pltpu.SemaphoreType.DMA,
        ],
        compiler_params=pltpu.CompilerParams(needs_layout_passes=False),
    )
    def hist_kernel(codes_hbm, out_hbm, buf0, buf1, lhist, merged, sem0, sem1):
        wid = lax.axis_index("s") * nc + lax.axis_index("c")
        base = wid * per_w

        zeros16 = jnp.zeros((lanes,), jnp.int32)
        zunroll = 8

        def zero_body(j, _):
            for u in range(zunroll):
                lhist[pl.ds(j * (lanes * zunroll) + u * lanes, lanes)] = zeros16
            return 0

        lax.fori_loop(0, (lanes * nbins) // (lanes * zunroll), zero_body, 0)

        ones = jnp.ones((lanes,), jnp.int32)
        lane_base = lax.iota(jnp.int32, lanes) * nbins

        bufs = (buf0, buf1)
        sems = (sem0, sem1)
        unroll = 8

        def consume(buf):
            def body(g, _):
                gbase = g * (lanes * unroll)
                for u in range(unroll):
                    v = buf[pl.ds(gbase + u * lanes, lanes)]
                    plsc.addupdate_scatter(lhist, [lane_base + v], ones)
                return 0

            lax.fori_loop(0, chunk // (lanes * unroll), body, 0)

        copies = [
            pltpu.async_copy(codes_hbm.at[pl.ds(base, chunk)], bufs[0], sems[0]),
            None,
        ]
        for i in range(nchunks):
            if i + 1 < nchunks:
                copies[(i + 1) % 2] = pltpu.async_copy(
                    codes_hbm.at[pl.ds(base + (i + 1) * chunk, chunk)],
                    bufs[(i + 1) % 2],
                    sems[(i + 1) % 2],
                )
            copies[i % 2].wait()
            consume(bufs[i % 2])

        for j in range(nbins // lanes):
            acc = jnp.zeros((lanes,), jnp.int32)
            for lane in range(lanes):
                acc = acc + lhist[pl.ds(lane * nbins + j * lanes, lanes)]
            merged[pl.ds(j * lanes, lanes)] = acc

        pltpu.sync_copy(merged, out_hbm.at[wid])

    return hist_kernel(codes)


def kernel(output, target):
    b, cls_num, h, w = output.shape
    nbins = cls_num * CODE_STRIDE
    codes = _argmax_pack(output, target)
    parts = _sc_hist(codes.reshape(-1), nbins)

    c = parts.sum(0).reshape(cls_num, CODE_STRIDE)[:, :cls_num]
    h_tgt = c.sum(1)
    h_out = c.sum(0)
    m = jnp.diagonal(c)
    tps = m.astype(jnp.float32)
    denom = (h_out + h_tgt - m).astype(jnp.float32)
    score = jnp.where(denom == 0, 0.0, tps / jnp.where(denom == 0, 1.0, denom))
    iou = score.mean()
    n = b * h * w
    total = b * cls_num * h * w
    t_sum = m.sum().astype(jnp.float32)
    pix_acc = (2.0 * t_sum - 2.0 * float(n) + float(total)) / float(total)
    return (iou, pix_acc)


# speedup vs baseline: 4.4440x; 1.1537x over previous
import functools

import jax
import jax.numpy as jnp
from jax import lax
from jax.experimental import pallas as pl
from jax.experimental.pallas import tpu as pltpu
from jax.experimental.pallas import tpu_sc as plsc

THRESH = 0.5
CODE_STRIDE = 32


def _argmax_pack_body(cls_num, out_ref, tgt_ref, code_ref):
    oidx = jnp.zeros(out_ref.shape[2:], jnp.int32)
    for c in range(cls_num - 1, -1, -1):
        oidx = jnp.where(out_ref[0, c] > THRESH, c, oidx)
    bestv = tgt_ref[0, 0]
    tidx = jnp.zeros(tgt_ref.shape[2:], jnp.int32)
    for c in range(1, cls_num):
        v = tgt_ref[0, c]
        m = v > bestv
        tidx = jnp.where(m, c, tidx)
        bestv = jnp.maximum(bestv, v)
    code_ref[0] = tidx * CODE_STRIDE + oidx


def _argmax_pack(output, target, hb=128):
    b, cls_num, h, w = output.shape
    grid = (b, h // hb)
    return pl.pallas_call(
        functools.partial(_argmax_pack_body, cls_num),
        grid=grid,
        in_specs=[
            pl.BlockSpec((1, cls_num, hb, w), lambda i, j: (i, 0, j, 0)),
            pl.BlockSpec((1, cls_num, hb, w), lambda i, j: (i, 0, j, 0)),
        ],
        out_specs=pl.BlockSpec((1, hb, w), lambda i, j: (i, j, 0)),
        out_shape=jax.ShapeDtypeStruct((b, h, w), jnp.int32),
        compiler_params=pltpu.CompilerParams(
            dimension_semantics=("parallel", "parallel"),
        ),
    )(output, target)


def _sc_hist(codes, nbins):
    info = plsc.get_sparse_core_info()
    nc, ns, lanes = info.num_cores, info.num_subcores, info.num_lanes
    nw = nc * ns
    b, h, w = codes.shape
    wpi = nw // b
    rows_pw = h // wpi
    crows = 32
    nchunks = rows_pw // crows
    groups_pr = w // lanes
    stride = nbins + 1

    mesh = plsc.VectorSubcoreMesh(core_axis_name="c", subcore_axis_name="s")

    @functools.partial(
        pl.kernel,
        mesh=mesh,
        out_type=jax.ShapeDtypeStruct((nw, nbins), jnp.int32),
        scratch_types=[
            pltpu.VMEM((crows, w), jnp.int32),
            pltpu.VMEM((crows, w), jnp.int32),
            pltpu.VMEM((lanes * stride,), jnp.int32),
            pltpu.VMEM((nbins,), jnp.int32),
            pltpu.SemaphoreType.DMA,
            pltpu.SemaphoreType.DMA,
        ],
        compiler_params=pltpu.CompilerParams(needs_layout_passes=False),
    )
    def hist_kernel(codes_hbm, out_hbm, buf0, buf1, lhist, merged, sem0, sem1):
        wid = lax.axis_index("s") * nc + lax.axis_index("c")
        img = wid // wpi
        row0 = (wid % wpi) * rows_pw

        zeros16 = jnp.zeros((lanes,), jnp.int32)
        zunroll = 8
        ztot = lanes * stride
        znum = ztot // (lanes * zunroll)

        def zero_body(j, _):
            for u in range(zunroll):
                lhist[pl.ds(j * (lanes * zunroll) + u * lanes, lanes)] = zeros16
            return 0

        lax.fori_loop(0, znum, zero_body, 0)
        for j in range(znum * zunroll, ztot // lanes):
            lhist[pl.ds(j * lanes, lanes)] = zeros16

        ones = jnp.ones((lanes,), jnp.int32)
        lane_base = lax.iota(jnp.int32, lanes) * stride

        bufs = (buf0, buf1)
        sems = (sem0, sem1)

        def consume(buf):
            def body(r, _):
                for g in range(groups_pr):
                    v = buf[r, pl.ds(g * lanes, lanes)]
                    plsc.addupdate_scatter(lhist, [lane_base + v], ones)
                return 0

            lax.fori_loop(0, crows, body, 0)

        copies = [
            pltpu.async_copy(
                codes_hbm.at[img, pl.ds(row0, crows)], bufs[0], sems[0]
            ),
            None,
        ]
        for i in range(nchunks):
            if i + 1 < nchunks:
                copies[(i + 1) % 2] = pltpu.async_copy(
                    codes_hbm.at[img, pl.ds(row0 + (i + 1) * crows, crows)],
                    bufs[(i + 1) % 2],
                    sems[(i + 1) % 2],
                )
            copies[i % 2].wait()
            consume(bufs[i % 2])

        for j in range(nbins // lanes):
            acc = jnp.zeros((lanes,), jnp.int32)
            for lane in range(lanes):
                acc = acc + lhist[pl.ds(lane * stride + j * lanes, lanes)]
            merged[pl.ds(j * lanes, lanes)] = acc

        pltpu.sync_copy(merged, out_hbm.at[wid])

    return hist_kernel(codes)


def kernel(output, target):
    b, cls_num, h, w = output.shape
    nbins = cls_num * CODE_STRIDE
    codes = _argmax_pack(output, target)
    parts = _sc_hist(codes, nbins)

    c = parts.sum(0).reshape(cls_num, CODE_STRIDE)[:, :cls_num]
    h_tgt = c.sum(1)
    h_out = c.sum(0)
    m = jnp.diagonal(c)
    tps = m.astype(jnp.float32)
    denom = (h_out + h_tgt - m).astype(jnp.float32)
    score = jnp.where(denom == 0, 0.0, tps / jnp.where(denom == 0, 1.0, denom))
    iou = score.mean()
    n = b * h * w
    total = b * cls_num * h * w
    t_sum = m.sum().astype(jnp.float32)
    pix_acc = (2.0 * t_sum - 2.0 * float(n) + float(total)) / float(total)
    return (iou, pix_acc)


# speedup vs baseline: 4.5867x; 1.0321x over previous
import functools

import jax
import jax.numpy as jnp
from jax import lax
from jax.experimental import pallas as pl
from jax.experimental.pallas import tpu as pltpu
from jax.experimental.pallas import tpu_sc as plsc

THRESH = 0.5
CODE_STRIDE = 32


def _argmax_pack_body(cls_num, out_ref, tgt_ref, code_ref):
    oidx = jnp.zeros(out_ref.shape[2:], jnp.int32)
    for c in range(cls_num - 1, -1, -1):
        oidx = jnp.where(out_ref[0, c] > THRESH, c, oidx)
    bestv = tgt_ref[0, 0]
    tidx = jnp.zeros(tgt_ref.shape[2:], jnp.int32)
    for c in range(1, cls_num):
        v = tgt_ref[0, c]
        m = v > bestv
        tidx = jnp.where(m, c, tidx)
        bestv = jnp.maximum(bestv, v)
    code_ref[0] = tidx * CODE_STRIDE + oidx


def _argmax_pack(output, target, boff, nb, hb=256):
    b, cls_num, h, w = output.shape
    grid = (nb, h // hb)
    return pl.pallas_call(
        functools.partial(_argmax_pack_body, cls_num),
        grid=grid,
        in_specs=[
            pl.BlockSpec((1, cls_num, hb, w), lambda i, j: (i + boff, 0, j, 0)),
            pl.BlockSpec((1, cls_num, hb, w), lambda i, j: (i + boff, 0, j, 0)),
        ],
        out_specs=pl.BlockSpec((1, hb, w), lambda i, j: (i, j, 0)),
        out_shape=jax.ShapeDtypeStruct((nb, h, w), jnp.int32),
        compiler_params=pltpu.CompilerParams(
            dimension_semantics=("parallel", "parallel"),
        ),
    )(output, target)


def _sc_hist(codes, nbins):
    info = plsc.get_sparse_core_info()
    nc, ns, lanes = info.num_cores, info.num_subcores, info.num_lanes
    nw = nc * ns
    b, h, w = codes.shape
    wpi = nw // b
    rows_pw = h // wpi
    crows = 32
    nchunks = rows_pw // crows
    groups_pr = w // lanes
    stride = nbins + 1

    mesh = plsc.VectorSubcoreMesh(core_axis_name="c", subcore_axis_name="s")

    @functools.partial(
        pl.kernel,
        mesh=mesh,
        out_type=jax.ShapeDtypeStruct((nw, nbins), jnp.int32),
        scratch_types=[
            pltpu.VMEM((crows, w), jnp.int32),
            pltpu.VMEM((crows, w), jnp.int32),
            pltpu.VMEM((lanes * stride,), jnp.int32),
            pltpu.VMEM((nbins,), jnp.int32),
            pltpu.SemaphoreType.DMA,
            pltpu.SemaphoreType.DMA,
        ],
        compiler_params=pltpu.CompilerParams(needs_layout_passes=False),
    )
    def hist_kernel(codes_hbm, out_hbm, buf0, buf1, lhist, merged, sem0, sem1):
        wid = lax.axis_index("s") * nc + lax.axis_index("c")
        img = wid // wpi
        row0 = (wid % wpi) * rows_pw

        zeros16 = jnp.zeros((lanes,), jnp.int32)
        zunroll = 8
        ztot = lanes * stride
        znum = ztot // (lanes * zunroll)

        def zero_body(j, _):
            for u in range(zunroll):
                lhist[pl.ds(j * (lanes * zunroll) + u * lanes, lanes)] = zeros16
            return 0

        lax.fori_loop(0, znum, zero_body, 0)
        for j in range(znum * zunroll, ztot // lanes):
            lhist[pl.ds(j * lanes, lanes)] = zeros16

        ones = jnp.ones((lanes,), jnp.int32)
        lane_base = lax.iota(jnp.int32, lanes) * stride

        bufs = (buf0, buf1)
        sems = (sem0, sem1)

        def consume(buf):
            def body(r, _):
                for g in range(groups_pr):
                    v = buf[r, pl.ds(g * lanes, lanes)]
                    plsc.addupdate_scatter(lhist, [lane_base + v], ones)
                return 0

            lax.fori_loop(0, crows, body, 0)

        copies = [
            pltpu.async_copy(
                codes_hbm.at[img, pl.ds(row0, crows)], bufs[0], sems[0]
            ),
            None,
        ]
        for i in range(nchunks):
            if i + 1 < nchunks:
                copies[(i + 1) % 2] = pltpu.async_copy(
                    codes_hbm.at[img, pl.ds(row0 + (i + 1) * crows, crows)],
                    bufs[(i + 1) % 2],
                    sems[(i + 1) % 2],
                )
            copies[i % 2].wait()
            consume(bufs[i % 2])

        for j in range(nbins // lanes):
            acc = jnp.zeros((lanes,), jnp.int32)
            for lane in range(lanes):
                acc = acc + lhist[pl.ds(lane * stride + j * lanes, lanes)]
            merged[pl.ds(j * lanes, lanes)] = acc

        pltpu.sync_copy(merged, out_hbm.at[wid])

    return hist_kernel(codes)


def kernel(output, target):
    b, cls_num, h, w = output.shape
    nbins = cls_num * CODE_STRIDE
    nsplit = 2
    nb = b // nsplit
    hist = None
    for s in range(nsplit):
        codes = _argmax_pack(output, target, s * nb, nb)
        parts = _sc_hist(codes, nbins)
        hsum = parts.sum(0)
        hist = hsum if hist is None else hist + hsum

    c = hist.reshape(cls_num, CODE_STRIDE)[:, :cls_num]
    h_tgt = c.sum(1)
    h_out = c.sum(0)
    m = jnp.diagonal(c)
    tps = m.astype(jnp.float32)
    denom = (h_out + h_tgt - m).astype(jnp.float32)
    score = jnp.where(denom == 0, 0.0, tps / jnp.where(denom == 0, 1.0, denom))
    iou = score.mean()
    n = b * h * w
    total = b * cls_num * h * w
    t_sum = m.sum().astype(jnp.float32)
    pix_acc = (2.0 * t_sum - 2.0 * float(n) + float(total)) / float(total)
    return (iou, pix_acc)
